# baseline (device time: 26558 ns/iter reference)
import jax
import jax.numpy as jnp
from jax import lax
from jax.experimental import pallas as pl
from jax.experimental.pallas import tpu as pltpu

N_DEV = 32
M_PER = 64


def kernel(x, w_mat):
    m_per, k_dim = x.shape
    _, n_dim = w_mat.shape
    tile = n_dim // N_DEV

    def body(x_ref, w_ref, out_ref, sbuf, rbuf, send_sems, recv_sems):
        me = lax.axis_index("i")

        barrier = pltpu.get_barrier_semaphore()
        for k in range(1, N_DEV):
            pl.semaphore_signal(
                barrier, inc=1,
                device_id=((me + k) % N_DEV,),
                device_id_type=pl.DeviceIdType.MESH,
            )
        pl.semaphore_wait(barrier, N_DEV - 1)

        y = jnp.dot(x_ref[...], w_ref[...], preferred_element_type=jnp.float32)
        y = jnp.maximum(y, 0.0)
        ybf = y.astype(jnp.bfloat16)

        for t in range(N_DEV):
            tile_v = ybf[:, t * tile:(t + 1) * tile]
            sbuf[t] = tile_v

            @pl.when(t == me)
            def _():
                out_ref[t * m_per:(t + 1) * m_per, :] = tile_v.astype(jnp.float32)

        rdmas = []
        for k in range(1, N_DEV):
            tgt = (me + k) % N_DEV
            rdma = pltpu.make_async_remote_copy(
                src_ref=sbuf.at[tgt],
                dst_ref=rbuf.at[k],
                send_sem=send_sems.at[k],
                recv_sem=recv_sems.at[k],
                device_id=(tgt,),
                device_id_type=pl.DeviceIdType.MESH,
            )
            rdma.start()
            rdmas.append(rdma)

        for k in range(1, N_DEV):
            rdmas[k - 1].wait_recv()
            src_dev = (me - k + N_DEV) % N_DEV
            out_ref[pl.ds(src_dev * m_per, m_per), :] = rbuf[k].astype(jnp.float32)

        for k in range(1, N_DEV):
            rdmas[k - 1].wait_send()

    return pl.pallas_call(
        body,
        out_shape=jax.ShapeDtypeStruct((n_dim, tile), jnp.float32),
        in_specs=[
            pl.BlockSpec(memory_space=pltpu.VMEM),
            pl.BlockSpec(memory_space=pltpu.VMEM),
        ],
        out_specs=pl.BlockSpec(memory_space=pltpu.VMEM),
        scratch_shapes=[
            pltpu.VMEM((N_DEV, m_per, tile), jnp.bfloat16),
            pltpu.VMEM((N_DEV, m_per, tile), jnp.bfloat16),
            pltpu.SemaphoreType.DMA((N_DEV,)),
            pltpu.SemaphoreType.DMA((N_DEV,)),
        ],
        compiler_params=pltpu.CompilerParams(collective_id=0),
    )(x, w_mat)


# device time: 23528 ns/iter; 1.1288x vs baseline; 1.1288x over previous
import jax
import jax.numpy as jnp
from jax import lax
from jax.experimental import pallas as pl
from jax.experimental.pallas import tpu as pltpu

N_DEV = 32
M_PER = 64


def kernel(x, w_mat):
    m_per, k_dim = x.shape
    _, n_dim = w_mat.shape
    tile = n_dim // N_DEV

    def body(x_ref, w_ref, out_ref, sbuf, rbuf, send_sems, recv_sems):
        me = lax.axis_index("i")

        barrier = pltpu.get_barrier_semaphore()
        for k in range(1, N_DEV):
            pl.semaphore_signal(
                barrier, inc=1,
                device_id=((me + k) % N_DEV,),
                device_id_type=pl.DeviceIdType.MESH,
            )

        y = jnp.dot(x_ref[...], w_ref[...], preferred_element_type=jnp.float32)
        y = jnp.maximum(y, 0.0)
        ybf = y.astype(jnp.bfloat16)

        for t in range(N_DEV):
            tile_v = ybf[:, t * tile:(t + 1) * tile]
            sbuf[t] = tile_v

            @pl.when(t == me)
            def _():
                rbuf[t] = tile_v

        pl.semaphore_wait(barrier, N_DEV - 1)

        rdmas = []
        for k in range(1, N_DEV):
            tgt = (me + k) % N_DEV
            rdma = pltpu.make_async_remote_copy(
                src_ref=sbuf.at[tgt],
                dst_ref=rbuf.at[me],
                send_sem=send_sems.at[k],
                recv_sem=recv_sems.at[me],
                device_id=(tgt,),
                device_id_type=pl.DeviceIdType.MESH,
            )
            rdma.start()
            rdmas.append(rdma)

        for k in range(1, N_DEV):
            src_dev = (me + k) % N_DEV
            recv = pltpu.make_async_remote_copy(
                src_ref=sbuf.at[src_dev],
                dst_ref=rbuf.at[src_dev],
                send_sem=send_sems.at[k],
                recv_sem=recv_sems.at[src_dev],
                device_id=(src_dev,),
                device_id_type=pl.DeviceIdType.MESH,
            )
            recv.wait_recv()

        out_ref[...] = rbuf[...].reshape(N_DEV * m_per, tile).astype(jnp.float32)

        for k in range(1, N_DEV):
            rdmas[k - 1].wait_send()

    return pl.pallas_call(
        body,
        out_shape=jax.ShapeDtypeStruct((n_dim, tile), jnp.float32),
        in_specs=[
            pl.BlockSpec(memory_space=pltpu.VMEM),
            pl.BlockSpec(memory_space=pltpu.VMEM),
        ],
        out_specs=pl.BlockSpec(memory_space=pltpu.VMEM),
        scratch_shapes=[
            pltpu.VMEM((N_DEV, m_per, tile), jnp.bfloat16),
            pltpu.VMEM((N_DEV, m_per, tile), jnp.bfloat16),
            pltpu.SemaphoreType.DMA((N_DEV,)),
            pltpu.SemaphoreType.DMA((N_DEV,)),
        ],
        compiler_params=pltpu.CompilerParams(collective_id=0),
    )(x, w_mat)
